# baseline (speedup 1.0000x reference)
import functools

import jax
import jax.numpy as jnp
from jax import lax
from jax.experimental import pallas as pl
from jax.experimental.pallas import tpu as pltpu
from jax.experimental.pallas import tpu_sc as plsc

_N = 100000
_NP = 100096
_E = 6400000
_LANES = 128
_NC = 2
_NS = 16
_NW = _NC * _NS
_KB = 8
_NOUT = 196
_RPT = _KB * _NOUT
_ROWS = _NW * _RPT
_EP = _ROWS * _LANES
_NPT = _NP // _NS
_NPAIR = _NOUT // 2
_R = _NP // 16

_OB = 1024

_sc_mesh = plsc.VectorSubcoreMesh(core_axis_name="c", subcore_axis_name="s")
_sc_params = pltpu.CompilerParams(use_tc_tiling_on_sc=False)


def _deg_body(dst_hbm, ones_hbm, o0_hbm, o1_hbm, dbuf, ones_v, deg_sh,
              isem, ssem):
    c = lax.axis_index("c")
    s = lax.axis_index("s")
    wid = s * _NC + c
    pltpu.sync_copy(ones_hbm, ones_v)
    for r in range(_NPT // _OB):
        pltpu.sync_copy(ones_v, deg_sh.at[pl.ds(s * _NPT + r * _OB, _OB)])
    _tail = _NPT - (_NPT // _OB) * _OB
    if _tail:
        pltpu.sync_copy(
            ones_v.at[pl.ds(0, _tail)],
            deg_sh.at[pl.ds(s * _NPT + (_NPT // _OB) * _OB, _tail)])
    plsc.subcore_barrier()

    base = wid * _RPT

    def idx_load(b, blk):
        return pltpu.async_copy(
            dst_hbm.at[pl.ds(base + blk * _KB, _KB)], dbuf.at[b], isem.at[b])

    idx_load(0, 0)
    idx_load(1, 1)

    def pair(p, carry):
        for b in range(2):
            blk = 2 * p + b
            pltpu.make_async_copy(
                dst_hbm.at[pl.ds(base + blk * _KB, _KB)], dbuf.at[b],
                isem.at[b]).wait()
            scat = [pltpu.async_copy(ones_v.at[pl.ds(0, _LANES)],
                                     deg_sh.at[dbuf.at[b, j]],
                                     ssem.at[b], add=True)
                    for j in range(_KB)]
            for d in scat:
                d.wait()

            @pl.when(p < _NPAIR - 1)
            def _():
                idx_load(b, blk + 2)
        return carry

    lax.fori_loop(0, _NPAIR, pair, 0)

    plsc.subcore_barrier()

    @pl.when(c == 0)
    def _():
        pltpu.sync_copy(deg_sh.at[pl.ds(s * _NPT, _NPT)],
                        o0_hbm.at[pl.ds(s * _NPT, _NPT)])

    @pl.when(c == 1)
    def _():
        pltpu.sync_copy(deg_sh.at[pl.ds(s * _NPT, _NPT)],
                        o1_hbm.at[pl.ds(s * _NPT, _NPT)])


@functools.partial(
    pl.kernel,
    mesh=_sc_mesh,
    out_type=[jax.ShapeDtypeStruct((_NP, 8), jnp.float32),
              jax.ShapeDtypeStruct((_NP, 8), jnp.float32)],
    compiler_params=_sc_params,
    scratch_types=[
        pltpu.VMEM((2, _KB, _LANES), jnp.int32),
        pltpu.VMEM((_OB, 8), jnp.float32),
        pltpu.VMEM_SHARED((_NP, 8), jnp.float32),
        pltpu.SemaphoreType.DMA((2,)),
        pltpu.SemaphoreType.DMA((2,)),
    ],
)
def _deg_call(dst_hbm, ones_hbm, o0_hbm, o1_hbm, dbuf, ones_v, deg_sh,
              isem, ssem):
    _deg_body(dst_hbm, ones_hbm, o0_hbm, o1_hbm, dbuf, ones_v, deg_sh,
              isem, ssem)


def _agg_body(src_hbm, dst_hbm, g_hbm, out_hbm, sbuf, dbuf, rows, acc, tab,
              isem, gsem, ssem):
    c = lax.axis_index("c")
    s = lax.axis_index("s")
    wid = s * _NC + c
    pltpu.sync_copy(g_hbm.at[pl.ds(s * _NPT, _NPT)],
                    tab.at[pl.ds(s * _NPT, _NPT)])
    pltpu.sync_copy(g_hbm.at[pl.ds(s * _NPT, _NPT)],
                    acc.at[pl.ds(s * _NPT, _NPT)])
    plsc.subcore_barrier()

    base = wid * _RPT

    def idx_load(b, blk):
        r0 = base + blk * _KB
        pltpu.async_copy(src_hbm.at[pl.ds(r0, _KB)], sbuf.at[b], isem.at[b])
        pltpu.async_copy(dst_hbm.at[pl.ds(r0, _KB)], dbuf.at[b], isem.at[b])

    def idx_wait(b, blk):
        r0 = base + blk * _KB
        pltpu.make_async_copy(
            src_hbm.at[pl.ds(r0, _KB)], sbuf.at[b], isem.at[b]).wait()
        pltpu.make_async_copy(
            dst_hbm.at[pl.ds(r0, _KB)], dbuf.at[b], isem.at[b]).wait()

    idx_load(0, 0)
    idx_load(1, 1)

    def pair(p, carry):
        for b in range(2):
            blk = 2 * p + b
            idx_wait(b, blk)
            gath = [pltpu.async_copy(tab.at[sbuf.at[b, j]], rows.at[b, j],
                                     gsem.at[j])
                    for j in range(_KB)]
            scat = []
            for j in range(_KB):
                gath[j].wait()
                scat.append(pltpu.async_copy(
                    rows.at[b, j], acc.at[dbuf.at[b, j]], ssem.at[b],
                    add=True))
            for d in scat:
                d.wait()

            @pl.when(p < _NPAIR - 1)
            def _():
                idx_load(b, blk + 2)
        return carry

    lax.fori_loop(0, _NPAIR, pair, 0)

    plsc.subcore_barrier()
    pltpu.sync_copy(acc.at[pl.ds(s * _NPT, _NPT)],
                    out_hbm.at[pl.ds(c * _NP + s * _NPT, _NPT)])


@functools.partial(
    pl.kernel,
    mesh=_sc_mesh,
    out_type=jax.ShapeDtypeStruct((_NC * _NP, 8), jnp.float32),
    compiler_params=_sc_params,
    scratch_types=[
        pltpu.VMEM((2, _KB, _LANES), jnp.int32),
        pltpu.VMEM((2, _KB, _LANES), jnp.int32),
        pltpu.VMEM((2, _KB, _LANES, 8), jnp.float32),
        pltpu.VMEM_SHARED((_NP, 8), jnp.float32),
        pltpu.VMEM_SHARED((_NP, 8), jnp.float32),
        pltpu.SemaphoreType.DMA((2,)),
        pltpu.SemaphoreType.DMA((_KB,)),
        pltpu.SemaphoreType.DMA((2,)),
    ],
)
def _agg_call(src_hbm, dst_hbm, g_hbm, out_hbm, sbuf, dbuf, rows, acc, tab,
              isem, gsem, ssem):
    _agg_body(src_hbm, dst_hbm, g_hbm, out_hbm, sbuf, dbuf, rows, acc, tab,
              isem, gsem, ssem)


_BR = 3128
_GG = _R // _BR


def _prep_body(x_ref, d0_ref, d1_ref, w_ref, g_ref, dinv_ref):
    deg = d0_ref[...] + d1_ref[...] - 1.0
    dinv = lax.rsqrt(deg)
    h = jnp.dot(x_ref[...], w_ref[...], preferred_element_type=jnp.float32)
    dinv_ref[...] = dinv
    g_ref[...] = dinv * h


def _prep_call(x128, d0, d1, Wbig):
    blk = pl.BlockSpec((_BR, 128), lambda i: (i, 0))
    return pl.pallas_call(
        _prep_body,
        grid=(_GG,),
        in_specs=[blk, blk, blk,
                  pl.BlockSpec((128, 128), lambda i: (0, 0))],
        out_specs=[blk, blk],
        out_shape=[jax.ShapeDtypeStruct((_R, 128), jnp.float32),
                   jax.ShapeDtypeStruct((_R, 128), jnp.float32)],
    )(x128, d0, d1, Wbig)


def _dense_body(a0_ref, a1_ref, g_ref, dinv_ref, w_ref, b_ref, gn_ref):
    s_val = dinv_ref[...] * (a0_ref[...] + a1_ref[...] - g_ref[...]) + b_ref[...]
    h = jnp.maximum(s_val, 0.0)
    gn_ref[...] = dinv_ref[...] * jnp.dot(
        h, w_ref[...], preferred_element_type=jnp.float32)


def _dense_call(acc2, g, dinv, Wbig, b128):
    blk = pl.BlockSpec((_BR, 128), lambda i: (i, 0))
    return pl.pallas_call(
        _dense_body,
        grid=(_GG,),
        in_specs=[
            pl.BlockSpec((_BR, 128), lambda i: (i, 0)),
            pl.BlockSpec((_BR, 128), lambda i: (i + _GG, 0)),
            blk, blk,
            pl.BlockSpec((128, 128), lambda i: (0, 0)),
            pl.BlockSpec((1, 128), lambda i: (0, 0)),
        ],
        out_specs=blk,
        out_shape=jax.ShapeDtypeStruct((_R, 128), jnp.float32),
    )(acc2, acc2, g, dinv, Wbig, b128)


def _final_body(a0_ref, a1_ref, g_ref, dinv_ref, b_ref, out_ref):
    out_ref[...] = (dinv_ref[...] * (a0_ref[...] + a1_ref[...] - g_ref[...])
                    + b_ref[...])


def _final_call(acc2, g, dinv, b128):
    blk = pl.BlockSpec((_BR, 128), lambda i: (i, 0))
    return pl.pallas_call(
        _final_body,
        grid=(_GG,),
        in_specs=[
            pl.BlockSpec((_BR, 128), lambda i: (i, 0)),
            pl.BlockSpec((_BR, 128), lambda i: (i + _GG, 0)),
            blk, blk,
            pl.BlockSpec((1, 128), lambda i: (0, 0)),
        ],
        out_specs=blk,
        out_shape=jax.ShapeDtypeStruct((_R, 128), jnp.float32),
    )(acc2, acc2, g, dinv, b128)


def _wbig(W):
    w8 = jnp.zeros((8, 8), jnp.float32).at[:W.shape[0], :4].set(W)
    return jnp.kron(jnp.eye(16, dtype=jnp.float32), w8)


def _b128(b):
    return jnp.tile(jnp.concatenate([b, jnp.zeros((4,), jnp.float32)]),
                    16).reshape(1, 128)


def kernel(x, edge_index, W1, b1, W2, b2, W3, b3):
    assert x.shape == (_N, 3) and edge_index.shape == (2, _E)
    ei = edge_index.astype(jnp.int32)
    dst = jnp.pad(ei[1], (0, _EP - _E),
                  constant_values=_N).reshape(_ROWS, _LANES)
    src = jnp.pad(ei[0], (0, _EP - _E),
                  constant_values=_N).reshape(_ROWS, _LANES)
    ones = jnp.ones((_OB, 8), jnp.float32)
    x128 = jnp.pad(x, ((0, _NP - _N), (0, 5))).reshape(_R, 128)

    d0, d1 = _deg_call(dst, ones)
    g1, dinv = _prep_call(x128, d0.reshape(_R, 128), d1.reshape(_R, 128),
                          _wbig(W1))

    acc = _agg_call(src, dst, g1.reshape(_NP, 8)).reshape(2 * _R, 128)
    g2 = _dense_call(acc, g1, dinv, _wbig(W2), _b128(b1))

    acc = _agg_call(src, dst, g2.reshape(_NP, 8)).reshape(2 * _R, 128)
    g3 = _dense_call(acc, g2, dinv, _wbig(W3), _b128(b2))

    acc = _agg_call(src, dst, g3.reshape(_NP, 8)).reshape(2 * _R, 128)
    out = _final_call(acc, g3, dinv, _b128(b3))
    return out.reshape(_NP, 8)[:_N, :4]


# speedup vs baseline: 1.0250x; 1.0250x over previous
import functools

import jax
import jax.numpy as jnp
from jax import lax
from jax.experimental import pallas as pl
from jax.experimental.pallas import tpu as pltpu
from jax.experimental.pallas import tpu_sc as plsc

_N = 100000
_NP = 100096
_E = 6400000
_LANES = 128
_NC = 2
_NS = 16
_NW = _NC * _NS
_KB = 8
_NOUT = 196
_RPT = _KB * _NOUT
_ROWS = _NW * _RPT
_EP = _ROWS * _LANES
_NPT = _NP // _NS
_NPAIR = _NOUT // 2
_R = _NP // 16

_OB = 1024

_sc_mesh = plsc.VectorSubcoreMesh(core_axis_name="c", subcore_axis_name="s")
_sc_params = pltpu.CompilerParams(use_tc_tiling_on_sc=False)


def _deg_body(dst_hbm, ones_hbm, o0_hbm, o1_hbm, dbuf, ones_v, deg_sh,
              isem, ssem):
    c = lax.axis_index("c")
    s = lax.axis_index("s")
    wid = s * _NC + c
    pltpu.sync_copy(ones_hbm, ones_v)
    for r in range(_NPT // _OB):
        pltpu.sync_copy(ones_v, deg_sh.at[pl.ds(s * _NPT + r * _OB, _OB)])
    _tail = _NPT - (_NPT // _OB) * _OB
    if _tail:
        pltpu.sync_copy(
            ones_v.at[pl.ds(0, _tail)],
            deg_sh.at[pl.ds(s * _NPT + (_NPT // _OB) * _OB, _tail)])
    plsc.subcore_barrier()

    base = wid * _RPT

    def idx_load(b, blk):
        return pltpu.async_copy(
            dst_hbm.at[pl.ds(base + blk * _KB, _KB)], dbuf.at[b], isem.at[b])

    idx_load(0, 0)
    idx_load(1, 1)

    def pair(p, carry):
        for b in range(2):
            blk = 2 * p + b
            pltpu.make_async_copy(
                dst_hbm.at[pl.ds(base + blk * _KB, _KB)], dbuf.at[b],
                isem.at[b]).wait()
            scat = [pltpu.async_copy(ones_v.at[pl.ds(0, _LANES)],
                                     deg_sh.at[dbuf.at[b, j]],
                                     ssem.at[b], add=True)
                    for j in range(_KB)]
            for d in scat:
                d.wait()

            @pl.when(p < _NPAIR - 1)
            def _():
                idx_load(b, blk + 2)
        return carry

    lax.fori_loop(0, _NPAIR, pair, 0)

    plsc.subcore_barrier()

    @pl.when(c == 0)
    def _():
        pltpu.sync_copy(deg_sh.at[pl.ds(s * _NPT, _NPT)],
                        o0_hbm.at[pl.ds(s * _NPT, _NPT)])

    @pl.when(c == 1)
    def _():
        pltpu.sync_copy(deg_sh.at[pl.ds(s * _NPT, _NPT)],
                        o1_hbm.at[pl.ds(s * _NPT, _NPT)])


@functools.partial(
    pl.kernel,
    mesh=_sc_mesh,
    out_type=[jax.ShapeDtypeStruct((_NP, 8), jnp.float32),
              jax.ShapeDtypeStruct((_NP, 8), jnp.float32)],
    compiler_params=_sc_params,
    scratch_types=[
        pltpu.VMEM((2, _KB, _LANES), jnp.int32),
        pltpu.VMEM((_OB, 8), jnp.float32),
        pltpu.VMEM_SHARED((_NP, 8), jnp.float32),
        pltpu.SemaphoreType.DMA((2,)),
        pltpu.SemaphoreType.DMA((2,)),
    ],
)
def _deg_call(dst_hbm, ones_hbm, o0_hbm, o1_hbm, dbuf, ones_v, deg_sh,
              isem, ssem):
    _deg_body(dst_hbm, ones_hbm, o0_hbm, o1_hbm, dbuf, ones_v, deg_sh,
              isem, ssem)


def _agg_body(src_hbm, dst_hbm, g_hbm, out_hbm, sbuf, dbuf, rows, acc, tab,
              isem, gsem, ssem):
    c = lax.axis_index("c")
    s = lax.axis_index("s")
    wid = s * _NC + c
    pltpu.sync_copy(g_hbm.at[pl.ds(s * _NPT, _NPT)],
                    tab.at[pl.ds(s * _NPT, _NPT)])
    pltpu.sync_copy(g_hbm.at[pl.ds(s * _NPT, _NPT)],
                    acc.at[pl.ds(s * _NPT, _NPT)])
    plsc.subcore_barrier()

    base = wid * _RPT

    def idx_load(b, blk):
        r0 = base + blk * _KB
        pltpu.async_copy(src_hbm.at[pl.ds(r0, _KB)], sbuf.at[b], isem.at[b])
        pltpu.async_copy(dst_hbm.at[pl.ds(r0, _KB)], dbuf.at[b], isem.at[b])

    def idx_wait(b, blk):
        r0 = base + blk * _KB
        pltpu.make_async_copy(
            src_hbm.at[pl.ds(r0, _KB)], sbuf.at[b], isem.at[b]).wait()
        pltpu.make_async_copy(
            dst_hbm.at[pl.ds(r0, _KB)], dbuf.at[b], isem.at[b]).wait()

    idx_load(0, 0)
    idx_load(1, 1)

    def pair(p, carry):
        for b in range(2):
            blk = 2 * p + b
            idx_wait(b, blk)
            gath = [pltpu.async_copy(tab.at[sbuf.at[b, j]], rows.at[b, j],
                                     gsem.at[j])
                    for j in range(_KB)]
            scat = []
            for j in range(_KB):
                gath[j].wait()
                scat.append(pltpu.async_copy(
                    rows.at[b, j], acc.at[dbuf.at[b, j]], ssem.at[b],
                    add=True))
            for d in scat:
                d.wait()

            @pl.when(p < _NPAIR - 1)
            def _():
                idx_load(b, blk + 2)
        return carry

    lax.fori_loop(0, _NPAIR, pair, 0)

    plsc.subcore_barrier()
    pltpu.sync_copy(acc.at[pl.ds(s * _NPT, _NPT)],
                    out_hbm.at[pl.ds(c * _NP + s * _NPT, _NPT)])


@functools.partial(
    pl.kernel,
    mesh=_sc_mesh,
    out_type=jax.ShapeDtypeStruct((_NC * _NP, 8), jnp.float32),
    compiler_params=_sc_params,
    scratch_types=[
        pltpu.VMEM((2, _KB, _LANES), jnp.int32),
        pltpu.VMEM((2, _KB, _LANES), jnp.int32),
        pltpu.VMEM((2, _KB, _LANES, 8), jnp.float32),
        pltpu.VMEM_SHARED((_NP, 8), jnp.float32),
        pltpu.VMEM_SHARED((_NP, 8), jnp.float32),
        pltpu.SemaphoreType.DMA((2,)),
        pltpu.SemaphoreType.DMA((_KB,)),
        pltpu.SemaphoreType.DMA((2,)),
    ],
)
def _agg_call(src_hbm, dst_hbm, g_hbm, out_hbm, sbuf, dbuf, rows, acc, tab,
              isem, gsem, ssem):
    _agg_body(src_hbm, dst_hbm, g_hbm, out_hbm, sbuf, dbuf, rows, acc, tab,
              isem, gsem, ssem)


_BR = 3128
_GG = _R // _BR


def _prep_body(x_ref, d0_ref, d1_ref, w_ref, g_ref, dinv_ref):
    deg = d0_ref[...] + d1_ref[...] - 1.0
    dinv = lax.rsqrt(deg)
    h = jnp.dot(x_ref[...], w_ref[...], preferred_element_type=jnp.float32)
    dinv_ref[...] = dinv
    g_ref[...] = dinv * h


def _prep_call(x128, d0, d1, Wbig):
    blk = pl.BlockSpec((_BR, 128), lambda i: (i, 0))
    return pl.pallas_call(
        _prep_body,
        grid=(_GG,),
        in_specs=[blk, blk, blk,
                  pl.BlockSpec((128, 128), lambda i: (0, 0))],
        out_specs=[blk, blk],
        out_shape=[jax.ShapeDtypeStruct((_R, 128), jnp.float32),
                   jax.ShapeDtypeStruct((_R, 128), jnp.float32)],
    )(x128, d0, d1, Wbig)


def _dense_body(a0_ref, a1_ref, g_ref, dinv_ref, w_ref, b_ref, gn_ref):
    s_val = dinv_ref[...] * (a0_ref[...] + a1_ref[...] - g_ref[...]) + b_ref[...]
    h = jnp.maximum(s_val, 0.0)
    gn_ref[...] = dinv_ref[...] * jnp.dot(
        h, w_ref[...], preferred_element_type=jnp.float32)


def _dense_call(acc2, g, dinv, Wbig, b128):
    blk = pl.BlockSpec((_BR, 128), lambda i: (i, 0))
    return pl.pallas_call(
        _dense_body,
        grid=(_GG,),
        in_specs=[
            pl.BlockSpec((_BR, 128), lambda i: (i, 0)),
            pl.BlockSpec((_BR, 128), lambda i: (i + _GG, 0)),
            blk, blk,
            pl.BlockSpec((128, 128), lambda i: (0, 0)),
            pl.BlockSpec((1, 128), lambda i: (0, 0)),
        ],
        out_specs=blk,
        out_shape=jax.ShapeDtypeStruct((_R, 128), jnp.float32),
    )(acc2, acc2, g, dinv, Wbig, b128)


def _final_body(a0_ref, a1_ref, g_ref, dinv_ref, b_ref, out_ref):
    out_ref[...] = (dinv_ref[...] * (a0_ref[...] + a1_ref[...] - g_ref[...])
                    + b_ref[...])


def _final_call(acc2, g, dinv, b128):
    blk = pl.BlockSpec((_BR, 128), lambda i: (i, 0))
    return pl.pallas_call(
        _final_body,
        grid=(_GG,),
        in_specs=[
            pl.BlockSpec((_BR, 128), lambda i: (i, 0)),
            pl.BlockSpec((_BR, 128), lambda i: (i + _GG, 0)),
            blk, blk,
            pl.BlockSpec((1, 128), lambda i: (0, 0)),
        ],
        out_specs=blk,
        out_shape=jax.ShapeDtypeStruct((_R, 128), jnp.float32),
    )(acc2, acc2, g, dinv, b128)


def _wbig(W):
    w8 = jnp.zeros((8, 8), jnp.float32).at[:W.shape[0], :4].set(W)
    return jnp.kron(jnp.eye(16, dtype=jnp.float32), w8)


def _b128(b):
    return jnp.tile(jnp.concatenate([b, jnp.zeros((4,), jnp.float32)]),
                    16).reshape(1, 128)


def kernel(x, edge_index, W1, b1, W2, b2, W3, b3):
    assert x.shape == (_N, 3) and edge_index.shape == (2, _E)
    ei = edge_index.astype(jnp.int32)
    ei = jnp.pad(ei, ((0, 0), (0, _EP - _E)), constant_values=_N)
    src = ei[0].reshape(_ROWS, _LANES)
    dst = ei[1].reshape(_ROWS, _LANES)
    ones = jnp.ones((_OB, 8), jnp.float32)
    x128 = jnp.pad(x, ((0, _NP - _N), (0, 5))).reshape(_R, 128)

    d0, d1 = _deg_call(dst, ones)
    g1, dinv = _prep_call(x128, d0.reshape(_R, 128), d1.reshape(_R, 128),
                          _wbig(W1))

    acc = _agg_call(src, dst, g1.reshape(_NP, 8)).reshape(2 * _R, 128)
    g2 = _dense_call(acc, g1, dinv, _wbig(W2), _b128(b1))

    acc = _agg_call(src, dst, g2.reshape(_NP, 8)).reshape(2 * _R, 128)
    g3 = _dense_call(acc, g2, dinv, _wbig(W3), _b128(b2))

    acc = _agg_call(src, dst, g3.reshape(_NP, 8)).reshape(2 * _R, 128)
    out = _final_call(acc, g3, dinv, _b128(b3))
    return out.reshape(_NP, 8)[:_N, :4]
